# Initial kernel scaffold; baseline (speedup 1.0000x reference)
import jax
import jax.numpy as jnp
from jax.experimental import pallas as pl


def kernel(x, y):
    raise NotImplementedError("write your pallas kernel here")


# speedup vs baseline: 1.0964x; 1.0964x over previous
import jax
import jax.numpy as jnp
from jax.experimental import pallas as pl


def _nn_body(r_ref, qT_ref, d_ref, i_ref):
    b = r_ref[0]
    aT = qT_ref[0]
    P2 = b.shape[0]
    b2 = jnp.sum(b * b, axis=1, keepdims=True)
    a2 = jnp.sum(aT * aT, axis=0, keepdims=True)
    cross = jnp.dot(b, aT)
    d2 = a2 + b2 - 2.0 * cross
    m = jnp.min(d2, axis=0, keepdims=True)
    row = jax.lax.broadcasted_iota(jnp.int32, d2.shape, 0)
    idx = jnp.min(jnp.where(d2 == m, row, P2), axis=0, keepdims=True)
    d_ref[0] = m
    i_ref[0] = idx


def _nn_dir(q, r, block_q=256):
    N, P1, D = q.shape
    P2 = r.shape[1]
    nb = P1 // block_q
    qT = jnp.swapaxes(q, 1, 2)
    dists, idx = pl.pallas_call(
        _nn_body,
        grid=(N, nb),
        in_specs=[
            pl.BlockSpec((1, P2, D), lambda n, i: (n, 0, 0)),
            pl.BlockSpec((1, D, block_q), lambda n, i: (n, 0, i)),
        ],
        out_specs=[
            pl.BlockSpec((1, 1, block_q), lambda n, i, nb=nb: (n * nb + i, 0, 0)),
            pl.BlockSpec((1, 1, block_q), lambda n, i, nb=nb: (n * nb + i, 0, 0)),
        ],
        out_shape=[
            jax.ShapeDtypeStruct((N * nb, 1, block_q), jnp.float32),
            jax.ShapeDtypeStruct((N * nb, 1, block_q), jnp.int32),
        ],
    )(r, qT)
    return dists.reshape(N, P1), idx.reshape(N, P1)


def kernel(x, y):
    cham_x, idx_x = _nn_dir(x, y)
    cham_y, idx_y = _nn_dir(y, x)
    return cham_x, cham_y, idx_x, idx_y
